# baseline (speedup 1.0000x reference)
import functools

import jax
import jax.numpy as jnp
from jax import lax
from jax.experimental import pallas as pl
from jax.experimental.pallas import tpu as pltpu
from jax.experimental.pallas import tpu_sc as plsc

N = 100000
E = 1600000
G = 64
DH = 128

NC = 2
NS = 16
NW = NC * NS

N_PAD = 100352
ROW_TILE = 512
N_TILES = N_PAD // ROW_TILE
SHARE_A = N_PAD // NS

E_PAD = 1605632
EW_A = E_PAD // NW
BA = 1792
NB_A = EW_A // BA
NF_A = BA // 128

ET_C = E_PAD // NS
BC = 2048
NB_C = ET_C // BC

CAP = 12544
NCHUNK = 8
SHARE_C = CAP // NS
SHARE_C_LAST = SHARE_C
TRASH = CAP

_f32 = jnp.float32
_i32 = jnp.int32


def _sc_mesh():
    return plsc.VectorSubcoreMesh(core_axis_name="c", subcore_axis_name="s")


def _agg1_body(x16, srch, dsth, zros, parts, acc, srcb, dstb, gdst2, rows2,
               semi, semg, sems):
    c = lax.axis_index("c")
    s = lax.axis_index("s")
    w = s * NC + c

    pltpu.sync_copy(zros, acc.at[pl.ds(s * SHARE_A, SHARE_A)])
    plsc.subcore_barrier()

    base = w * EW_A
    pltpu.async_copy(srch.at[pl.ds(base, BA)], srcb.at[pl.ds(0, BA)], semi)
    pltpu.async_copy(dsth.at[pl.ds(base, BA)], dstb.at[pl.ds(0, BA)], semi)

    def batch(b, _):
        sl = (b % 2) * BA
        pltpu.make_async_copy(srch.at[pl.ds(base, BA)],
                              srcb.at[pl.ds(0, BA)], semi).wait()
        pltpu.make_async_copy(dsth.at[pl.ds(base, BA)],
                              dstb.at[pl.ds(0, BA)], semi).wait()
        gdst2b = gdst2

        @pl.when(b + 1 < NB_A)
        def _():
            off2 = base + (b + 1) * BA
            sl2 = ((b + 1) % 2) * BA
            pltpu.async_copy(srch.at[pl.ds(off2, BA)],
                             srcb.at[pl.ds(sl2, BA)], semi)
            pltpu.async_copy(dsth.at[pl.ds(off2, BA)],
                             dstb.at[pl.ds(sl2, BA)], semi)

        pltpu.async_copy(x16.at[srcb.at[pl.ds(sl, 128)]],
                         rows2.at[pl.ds(0, 128)], semg)

        def flush(j, _):
            rsl = (j % 2) * 128
            pltpu.make_async_copy(x16.at[srcb.at[pl.ds(sl, 128)]],
                                  rows2.at[pl.ds(0, 128)], semg).wait()

            @pl.when(j + 1 < NF_A)
            def _():
                pltpu.async_copy(
                    x16.at[srcb.at[pl.ds(sl + (j + 1) * 128, 128)]],
                    rows2.at[pl.ds(((j + 1) % 2) * 128, 128)], semg)

            @pl.when(j >= 1)
            def _():
                pltpu.make_async_copy(rows2.at[pl.ds(0, 128)],
                                      acc.at[gdst2b.at[0]], sems).wait()

            for k in range(8):
                gdst2b[j % 2, pl.ds(k * 16, 16)] = \
                    dstb[pl.ds(sl + j * 128 + k * 16, 16)]
            pltpu.async_copy(rows2.at[pl.ds(rsl, 128)],
                             acc.at[gdst2b.at[j % 2]], sems, add=True)
            return 0

        lax.fori_loop(0, NF_A, flush, 0)
        pltpu.make_async_copy(rows2.at[pl.ds(0, 128)],
                              acc.at[gdst2b.at[0]], sems).wait()
        return 0

    lax.fori_loop(0, NB_A, batch, 0)
    plsc.subcore_barrier()
    pltpu.sync_copy(acc.at[pl.ds(s * SHARE_A, SHARE_A)],
                    parts.at[c, pl.ds(s * SHARE_A, SHARE_A)])


def _agg1(x16, src, dst, zrosA):
    k = pl.kernel(
        _agg1_body,
        out_type=jax.ShapeDtypeStruct((NC, N_PAD, 16), _f32),
        mesh=_sc_mesh(),
        compiler_params=pltpu.CompilerParams(use_tc_tiling_on_sc=False),
        scratch_types=[
            pltpu.VMEM_SHARED((N_PAD, 16), _f32),
            pltpu.VMEM((2 * BA,), _i32),
            pltpu.VMEM((2 * BA,), _i32),
            pltpu.VMEM((2, 128), _i32),
            pltpu.VMEM((256, 16), _f32),
            pltpu.SemaphoreType.DMA,
            pltpu.SemaphoreType.DMA,
            pltpu.SemaphoreType.DMA,
        ],
    )
    return k(x16, src, dst, zrosA)


def _agg2_body(h1, srch, dsth, zros, agg2, acc, srcb, dstb, st_s, st_d2,
               rows2, semi, semg, sems):
    c = lax.axis_index("c")
    s = lax.axis_index("s")
    w = s * NC + c
    padsrc = w * 128 + lax.iota(_i32, 16) * 8
    trashv = jnp.full((16,), TRASH, _i32)
    base = s * ET_C

    for p in range(NCHUNK // NC):
        ch = NC * p + c
        lo = ch * CAP

        pltpu.sync_copy(zros, acc.at[pl.ds(s * SHARE_C, SHARE_C)])
        plsc.subcore_barrier()

        pltpu.async_copy(srch.at[pl.ds(base, BC)], srcb.at[pl.ds(0, BC)], semi)
        pltpu.async_copy(dsth.at[pl.ds(base, BC)], dstb.at[pl.ds(0, BC)], semi)

        def flush(j, nf):
            pltpu.make_async_copy(h1.at[st_s.at[pl.ds(0, 128)]],
                                  rows2.at[pl.ds(0, 128)], semg).wait()
            pltpu.sync_copy(rows2.at[pl.ds(0, 128)],
                            acc.at[st_d2.at[j]], add=True)

            @pl.when(j + 1 < nf)
            def _():
                pltpu.async_copy(
                    h1.at[st_s.at[pl.ds((j + 1) * 128, 128)]],
                    rows2.at[pl.ds(0, 128)], semg)

        def batch(b, remv):
            sl = (b % 2) * BC
            pltpu.make_async_copy(srch.at[pl.ds(base, BC)],
                                  srcb.at[pl.ds(0, BC)], semi).wait()
            pltpu.make_async_copy(dsth.at[pl.ds(base, BC)],
                                  dstb.at[pl.ds(0, BC)], semi).wait()

            @pl.when(b + 1 < NB_C)
            def _():
                off2 = base + (b + 1) * BC
                sl2 = ((b + 1) % 2) * BC
                pltpu.async_copy(srch.at[pl.ds(off2, BC)],
                                 srcb.at[pl.ds(sl2, BC)], semi)
                pltpu.async_copy(dsth.at[pl.ds(off2, BC)],
                                 dstb.at[pl.ds(sl2, BC)], semi)

            def grp(g, ptrv):
                d = dstb[pl.ds(sl + g * 16, 16)]
                si = srcb[pl.ds(sl + g * 16, 16)]
                m = (d >= lo) & (d < lo + CAP)
                mi = jnp.where(m, 1, 0).astype(_i32)
                csum = plsc.cumsum(mi)
                pos = ptrv + csum - mi
                plsc.store_scatter(st_s, [pos], si, mask=m)
                plsc.store_scatter(st_d2, [pos >> 7, pos & 127], d - lo,
                                   mask=m)
                return ptrv + plsc.all_reduce_population_count(m)

            ptrv = lax.fori_loop(0, BC // 16, grp, remv, unroll=4)
            ptr = jnp.max(ptrv)
            nf = ptr // 128

            @pl.when(nf > 0)
            def _():
                pltpu.async_copy(h1.at[st_s.at[pl.ds(0, 128)]],
                                 rows2.at[pl.ds(0, 128)], semg)

            def fl(j, _):
                flush(j, nf)
                return 0

            lax.fori_loop(0, nf, fl, 0)

            nrem = ptr - nf * 128
            for k in range(8):
                @pl.when((16 * k < nrem) & (nf > 0))
                def _():
                    st_s[pl.ds(16 * k, 16)] = st_s[pl.ds(nf * 128 + 16 * k, 16)]
                    row = st_d2[nf, pl.ds(16 * k, 16)]
                    st_d2[0, pl.ds(16 * k, 16)] = row
            return ptrv - nf * 128

        remv = lax.fori_loop(0, NB_C, batch, jnp.zeros((16,), _i32))
        rem = jnp.max(remv)

        for k in range(8):
            @pl.when(rem + 16 * k < 128)
            def _():
                pp = rem + 16 * k + lax.iota(_i32, 16)
                st_s[pl.ds(rem + 16 * k, 16)] = padsrc
                plsc.store_scatter(st_d2, [pp >> 7, pp & 127], trashv)

        @pl.when(rem > 0)
        def _():
            pltpu.async_copy(h1.at[st_s.at[pl.ds(0, 128)]],
                             rows2.at[pl.ds(0, 128)], semg)
            pltpu.make_async_copy(h1.at[st_s.at[pl.ds(0, 128)]],
                                  rows2.at[pl.ds(0, 128)], semg).wait()
            pltpu.sync_copy(rows2.at[pl.ds(0, 128)], acc.at[st_d2.at[0]],
                            add=True)

        plsc.subcore_barrier()

        pltpu.sync_copy(acc.at[pl.ds(s * SHARE_C, SHARE_C)],
                        agg2.at[pl.ds(lo + s * SHARE_C, SHARE_C)])
        plsc.subcore_barrier()


def _agg2(h1, src, dst, zrosC):
    k = pl.kernel(
        _agg2_body,
        out_type=jax.ShapeDtypeStruct((N_PAD, DH), _f32),
        mesh=_sc_mesh(),
        compiler_params=pltpu.CompilerParams(needs_layout_passes=False),
        scratch_types=[
            pltpu.VMEM_SHARED((CAP + 16, DH), _f32),
            pltpu.VMEM((2 * BC,), _i32),
            pltpu.VMEM((2 * BC,), _i32),
            pltpu.VMEM((2176,), _i32),
            pltpu.VMEM((17, 128), _i32),
            pltpu.VMEM((128, DH), _f32),
            pltpu.SemaphoreType.DMA,
            pltpu.SemaphoreType.DMA,
            pltpu.SemaphoreType.DMA,
        ],
    )
    return k(h1, src, dst, zrosC)


def _h1_body(parts, x16, w1, wr, b1, out):
    a = parts[0] + parts[1]
    acc = jnp.dot(a, w1[...], preferred_element_type=_f32,
                  precision=lax.Precision.HIGHEST)
    acc += jnp.dot(x16[...], wr[...], preferred_element_type=_f32,
                   precision=lax.Precision.HIGHEST)
    out[...] = jnp.maximum(acc + b1[...], 0.0)


def _h1_call(parts, x16, w1p, wrp, b1r):
    return pl.pallas_call(
        _h1_body,
        grid=(N_TILES,),
        in_specs=[
            pl.BlockSpec((NC, ROW_TILE, 16), lambda i: (0, i, 0)),
            pl.BlockSpec((ROW_TILE, 16), lambda i: (i, 0)),
            pl.BlockSpec((16, DH), lambda i: (0, 0)),
            pl.BlockSpec((16, DH), lambda i: (0, 0)),
            pl.BlockSpec((1, DH), lambda i: (0, 0)),
        ],
        out_specs=pl.BlockSpec((ROW_TILE, DH), lambda i: (i, 0)),
        out_shape=jax.ShapeDtypeStruct((N_PAD, DH), _f32),
    )(parts, x16, w1p, wrp, b1r)


def _out_body(a2, h1r, bidr, w2, wr2, b2, wfc, bfc, out, accp, accc):
    i = pl.program_id(0)

    @pl.when(i == 0)
    def _():
        accp[...] = jnp.zeros((G, DH), _f32)
        accc[...] = jnp.zeros((G, DH), _f32)

    h2 = jnp.dot(a2[...], w2[...], preferred_element_type=_f32,
                 precision=lax.Precision.HIGHEST)
    h2 += jnp.dot(h1r[...], wr2[...], preferred_element_type=_f32,
                  precision=lax.Precision.HIGHEST)
    h2 = jnp.maximum(h2 + b2[...], 0.0)

    bid = bidr[...][0, 0]
    row_limit = jnp.where(i == N_TILES - 1, N - (N_TILES - 1) * ROW_TILE,
                          ROW_TILE)
    h2 = jnp.where(lax.broadcasted_iota(_i32, (ROW_TILE, DH), 0) < row_limit,
                   h2, 0.0)
    oh = (lax.broadcasted_iota(_i32, (G, ROW_TILE), 0) == bid[None, :]).astype(_f32)
    accp[...] += jnp.dot(oh, h2, preferred_element_type=_f32,
                         precision=lax.Precision.HIGHEST)
    accc[...] += jnp.broadcast_to(jnp.sum(oh, axis=1, keepdims=True), (G, DH))

    @pl.when(i == N_TILES - 1)
    def _():
        pooled = accp[...] / jnp.maximum(accc[...], 1.0)
        z = jnp.dot(pooled, wfc[...], preferred_element_type=_f32,
                    precision=lax.Precision.HIGHEST) + bfc[...]
        out[...] = jax.nn.sigmoid(z)


def _out_call(agg2, h1, batch3, w2, wr2, b2r, wfcp, bfcp):
    return pl.pallas_call(
        _out_body,
        grid=(N_TILES,),
        in_specs=[
            pl.BlockSpec((ROW_TILE, DH), lambda i: (i, 0)),
            pl.BlockSpec((ROW_TILE, DH), lambda i: (i, 0)),
            pl.BlockSpec((1, 1, ROW_TILE), lambda i: (i, 0, 0)),
            pl.BlockSpec((DH, DH), lambda i: (0, 0)),
            pl.BlockSpec((DH, DH), lambda i: (0, 0)),
            pl.BlockSpec((1, DH), lambda i: (0, 0)),
            pl.BlockSpec((DH, 256), lambda i: (0, 0)),
            pl.BlockSpec((1, 256), lambda i: (0, 0)),
        ],
        out_specs=pl.BlockSpec((G, 256), lambda i: (0, 0)),
        out_shape=jax.ShapeDtypeStruct((G, 256), _f32),
        scratch_shapes=[
            pltpu.VMEM((G, DH), _f32),
            pltpu.VMEM((G, DH), _f32),
        ],
    )(agg2, h1, batch3, w2, wr2, b2r, wfcp, bfcp)


@jax.jit
def kernel(x, edge_index, batch, W_rel1, b_rel1, W_root1, W_rel2, b_rel2,
           W_root2, W_fc, b_fc):
    src = edge_index[0]
    dst = edge_index[1]
    npad = E_PAD - E
    src_p = jnp.concatenate([src, (jnp.arange(npad, dtype=_i32) * 17) % N])
    dst_p = jnp.concatenate([dst, jnp.full((npad,), N, _i32)])

    x16 = jnp.zeros((N_PAD, 16), _f32).at[:N, :3].set(x)
    w1p = jnp.zeros((16, DH), _f32).at[:3].set(W_rel1)
    wrp = jnp.zeros((16, DH), _f32).at[:3].set(W_root1)
    b1r = b_rel1.reshape(1, DH)
    b2r = b_rel2.reshape(1, DH)
    wfcp = jnp.zeros((DH, 256), _f32).at[:, :138].set(W_fc)
    bfcp = jnp.zeros((1, 256), _f32).at[0, :138].set(b_fc)
    batch3 = jnp.concatenate([batch, jnp.full((N_PAD - N,), G, _i32)])
    batch3 = batch3.reshape(N_TILES, 1, ROW_TILE)
    zrosA = jnp.zeros((SHARE_A, 16), _f32)
    zrosC = jnp.zeros((SHARE_C, DH), _f32)

    parts = _agg1(x16, src_p, dst_p, zrosA)
    h1 = _h1_call(parts, x16, w1p, wrp, b1r)
    agg2 = _agg2(h1, src_p, dst_p, zrosC)
    outp = _out_call(agg2, h1, batch3, W_rel2, W_root2, b2r, wfcp, bfcp)
    return outp[:, :138]


# speedup vs baseline: 1.0779x; 1.0779x over previous
import functools

import jax
import jax.numpy as jnp
from jax import lax
from jax.experimental import pallas as pl
from jax.experimental.pallas import tpu as pltpu
from jax.experimental.pallas import tpu_sc as plsc

N = 100000
E = 1600000
G = 64
DH = 128

NC = 2
NS = 16
NW = NC * NS

N_PAD = 100352
ROW_TILE = 512
N_TILES = N_PAD // ROW_TILE
SHARE_A = N_PAD // NS

E_PAD = 1605632
EW_A = E_PAD // NW
BA = 1792
NB_A = EW_A // BA
NF_A = BA // 128

ET_C = E_PAD // NS
BC = 2048
NB_C = ET_C // BC

CAP = 10000
NCHUNK = 10
SHARE_C = 632
SHARE_C_LAST = CAP - 15 * SHARE_C
TRASH = CAP

_f32 = jnp.float32
_i32 = jnp.int32


def _sc_mesh():
    return plsc.VectorSubcoreMesh(core_axis_name="c", subcore_axis_name="s")


def _agg1_body(x16, srch, dsth, zros, parts, acc, srcb, dstb, gdst2, rows2,
               semi, semg, sems):
    c = lax.axis_index("c")
    s = lax.axis_index("s")
    w = s * NC + c

    pltpu.sync_copy(zros, acc.at[pl.ds(s * SHARE_A, SHARE_A)])
    plsc.subcore_barrier()

    base = w * EW_A
    pltpu.async_copy(srch.at[pl.ds(base, BA)], srcb.at[pl.ds(0, BA)], semi)
    pltpu.async_copy(dsth.at[pl.ds(base, BA)], dstb.at[pl.ds(0, BA)], semi)

    def batch(b, _):
        sl = (b % 2) * BA
        pltpu.make_async_copy(srch.at[pl.ds(base, BA)],
                              srcb.at[pl.ds(0, BA)], semi).wait()
        pltpu.make_async_copy(dsth.at[pl.ds(base, BA)],
                              dstb.at[pl.ds(0, BA)], semi).wait()
        gdst2b = gdst2

        @pl.when(b + 1 < NB_A)
        def _():
            off2 = base + (b + 1) * BA
            sl2 = ((b + 1) % 2) * BA
            pltpu.async_copy(srch.at[pl.ds(off2, BA)],
                             srcb.at[pl.ds(sl2, BA)], semi)
            pltpu.async_copy(dsth.at[pl.ds(off2, BA)],
                             dstb.at[pl.ds(sl2, BA)], semi)

        pltpu.async_copy(x16.at[srcb.at[pl.ds(sl, 128)]],
                         rows2.at[pl.ds(0, 128)], semg)

        def flush(j, _):
            rsl = (j % 2) * 128
            pltpu.make_async_copy(x16.at[srcb.at[pl.ds(sl, 128)]],
                                  rows2.at[pl.ds(0, 128)], semg).wait()

            @pl.when(j + 1 < NF_A)
            def _():
                pltpu.async_copy(
                    x16.at[srcb.at[pl.ds(sl + (j + 1) * 128, 128)]],
                    rows2.at[pl.ds(((j + 1) % 2) * 128, 128)], semg)

            @pl.when(j >= 1)
            def _():
                pltpu.make_async_copy(rows2.at[pl.ds(0, 128)],
                                      acc.at[gdst2b.at[0]], sems).wait()

            for k in range(8):
                gdst2b[j % 2, pl.ds(k * 16, 16)] = \
                    dstb[pl.ds(sl + j * 128 + k * 16, 16)]
            pltpu.async_copy(rows2.at[pl.ds(rsl, 128)],
                             acc.at[gdst2b.at[j % 2]], sems, add=True)
            return 0

        lax.fori_loop(0, NF_A, flush, 0)
        pltpu.make_async_copy(rows2.at[pl.ds(0, 128)],
                              acc.at[gdst2b.at[0]], sems).wait()
        return 0

    lax.fori_loop(0, NB_A, batch, 0)
    plsc.subcore_barrier()
    pltpu.sync_copy(acc.at[pl.ds(s * SHARE_A, SHARE_A)],
                    parts.at[c, pl.ds(s * SHARE_A, SHARE_A)])


def _agg1(x16, src, dst, zrosA):
    k = pl.kernel(
        _agg1_body,
        out_type=jax.ShapeDtypeStruct((NC, N_PAD, 16), _f32),
        mesh=_sc_mesh(),
        compiler_params=pltpu.CompilerParams(use_tc_tiling_on_sc=False),
        scratch_types=[
            pltpu.VMEM_SHARED((N_PAD, 16), _f32),
            pltpu.VMEM((2 * BA,), _i32),
            pltpu.VMEM((2 * BA,), _i32),
            pltpu.VMEM((2, 128), _i32),
            pltpu.VMEM((256, 16), _f32),
            pltpu.SemaphoreType.DMA,
            pltpu.SemaphoreType.DMA,
            pltpu.SemaphoreType.DMA,
        ],
    )
    return k(x16, src, dst, zrosA)


def _agg2_body(h1, srch, dsth, zros, agg2, acc, srcb, dstb, st_s, st_d2,
               rows2, semi, semg, sems):
    c = lax.axis_index("c")
    s = lax.axis_index("s")
    w = s * NC + c
    padsrc = w * 128 + lax.iota(_i32, 16) * 8
    trashv = jnp.full((16,), TRASH, _i32)
    base = s * ET_C

    for p in range(NCHUNK // NC):
        ch = NC * p + c
        lo = ch * CAP

        @pl.when(s < NS - 1)
        def _():
            pltpu.sync_copy(zros, acc.at[pl.ds(s * SHARE_C, SHARE_C)])

        @pl.when(s == NS - 1)
        def _():
            pltpu.sync_copy(zros.at[pl.ds(0, SHARE_C_LAST)],
                            acc.at[pl.ds(s * SHARE_C, SHARE_C_LAST)])

        plsc.subcore_barrier()

        pltpu.async_copy(srch.at[pl.ds(base, BC)], srcb.at[pl.ds(0, BC)], semi)
        pltpu.async_copy(dsth.at[pl.ds(base, BC)], dstb.at[pl.ds(0, BC)], semi)

        def flush(j, nf):
            rsl = (j % 2) * 128
            pltpu.make_async_copy(h1.at[st_s.at[pl.ds(0, 128)]],
                                  rows2.at[pl.ds(0, 128)], semg).wait()

            @pl.when(j >= 1)
            def _():
                pltpu.make_async_copy(rows2.at[pl.ds(0, 128)],
                                      acc.at[st_d2.at[0]], sems).wait()

            @pl.when(j + 1 < nf)
            def _():
                pltpu.async_copy(
                    h1.at[st_s.at[pl.ds((j + 1) * 128, 128)]],
                    rows2.at[pl.ds(((j + 1) % 2) * 128, 128)], semg)

            pltpu.async_copy(rows2.at[pl.ds(rsl, 128)],
                             acc.at[st_d2.at[j]], sems, add=True)

        def batch(b, remv):
            sl = (b % 2) * BC
            pltpu.make_async_copy(srch.at[pl.ds(base, BC)],
                                  srcb.at[pl.ds(0, BC)], semi).wait()
            pltpu.make_async_copy(dsth.at[pl.ds(base, BC)],
                                  dstb.at[pl.ds(0, BC)], semi).wait()

            @pl.when(b + 1 < NB_C)
            def _():
                off2 = base + (b + 1) * BC
                sl2 = ((b + 1) % 2) * BC
                pltpu.async_copy(srch.at[pl.ds(off2, BC)],
                                 srcb.at[pl.ds(sl2, BC)], semi)
                pltpu.async_copy(dsth.at[pl.ds(off2, BC)],
                                 dstb.at[pl.ds(sl2, BC)], semi)

            def grp(g, ptrv):
                d = dstb[pl.ds(sl + g * 16, 16)]
                si = srcb[pl.ds(sl + g * 16, 16)]
                m = (d >= lo) & (d < lo + CAP)
                mi = jnp.where(m, 1, 0).astype(_i32)
                csum = plsc.cumsum(mi)
                pos = ptrv + csum - mi
                plsc.store_scatter(st_s, [pos], si, mask=m)
                plsc.store_scatter(st_d2, [pos >> 7, pos & 127], d - lo,
                                   mask=m)
                return ptrv + plsc.all_reduce_population_count(m)

            ptrv = lax.fori_loop(0, BC // 16, grp, remv, unroll=4)
            ptr = jnp.max(ptrv)
            nf = ptr // 128

            @pl.when(nf > 0)
            def _():
                pltpu.async_copy(h1.at[st_s.at[pl.ds(0, 128)]],
                                 rows2.at[pl.ds(0, 128)], semg)

            def fl(j, _):
                flush(j, nf)
                return 0

            lax.fori_loop(0, nf, fl, 0)

            @pl.when(nf > 0)
            def _():
                pltpu.make_async_copy(rows2.at[pl.ds(0, 128)],
                                      acc.at[st_d2.at[0]], sems).wait()

            nrem = ptr - nf * 128
            for k in range(8):
                @pl.when((16 * k < nrem) & (nf > 0))
                def _():
                    st_s[pl.ds(16 * k, 16)] = st_s[pl.ds(nf * 128 + 16 * k, 16)]
                    row = st_d2[nf, pl.ds(16 * k, 16)]
                    st_d2[0, pl.ds(16 * k, 16)] = row
            return ptrv - nf * 128

        remv = lax.fori_loop(0, NB_C, batch, jnp.zeros((16,), _i32))
        rem = jnp.max(remv)

        for k in range(8):
            @pl.when(rem + 16 * k < 128)
            def _():
                pp = rem + 16 * k + lax.iota(_i32, 16)
                st_s[pl.ds(rem + 16 * k, 16)] = padsrc
                plsc.store_scatter(st_d2, [pp >> 7, pp & 127], trashv)

        @pl.when(rem > 0)
        def _():
            pltpu.async_copy(h1.at[st_s.at[pl.ds(0, 128)]],
                             rows2.at[pl.ds(0, 128)], semg)
            pltpu.make_async_copy(h1.at[st_s.at[pl.ds(0, 128)]],
                                  rows2.at[pl.ds(0, 128)], semg).wait()
            pltpu.sync_copy(rows2.at[pl.ds(0, 128)], acc.at[st_d2.at[0]],
                            add=True)

        plsc.subcore_barrier()

        @pl.when(s < NS - 1)
        def _():
            pltpu.sync_copy(acc.at[pl.ds(s * SHARE_C, SHARE_C)],
                            agg2.at[pl.ds(lo + s * SHARE_C, SHARE_C)])

        @pl.when(s == NS - 1)
        def _():
            pltpu.sync_copy(acc.at[pl.ds(s * SHARE_C, SHARE_C_LAST)],
                            agg2.at[pl.ds(lo + s * SHARE_C, SHARE_C_LAST)])

        plsc.subcore_barrier()


def _agg2(h1, src, dst, zrosC):
    k = pl.kernel(
        _agg2_body,
        out_type=jax.ShapeDtypeStruct((N_PAD, DH), _f32),
        mesh=_sc_mesh(),
        compiler_params=pltpu.CompilerParams(needs_layout_passes=False),
        scratch_types=[
            pltpu.VMEM_SHARED((CAP + 16, DH), _f32),
            pltpu.VMEM((2 * BC,), _i32),
            pltpu.VMEM((2 * BC,), _i32),
            pltpu.VMEM((2176,), _i32),
            pltpu.VMEM((17, 128), _i32),
            pltpu.VMEM((256, DH), _f32),
            pltpu.SemaphoreType.DMA,
            pltpu.SemaphoreType.DMA,
            pltpu.SemaphoreType.DMA,
        ],
    )
    return k(h1, src, dst, zrosC)


def _h1_body(parts, x16, w1, wr, b1, out):
    a = parts[0] + parts[1]
    acc = jnp.dot(a, w1[...], preferred_element_type=_f32,
                  precision=None)
    acc += jnp.dot(x16[...], wr[...], preferred_element_type=_f32,
                   precision=None)
    out[...] = jnp.maximum(acc + b1[...], 0.0)


def _h1_call(parts, x16, w1p, wrp, b1r):
    return pl.pallas_call(
        _h1_body,
        grid=(N_TILES,),
        in_specs=[
            pl.BlockSpec((NC, ROW_TILE, 16), lambda i: (0, i, 0)),
            pl.BlockSpec((ROW_TILE, 16), lambda i: (i, 0)),
            pl.BlockSpec((16, DH), lambda i: (0, 0)),
            pl.BlockSpec((16, DH), lambda i: (0, 0)),
            pl.BlockSpec((1, DH), lambda i: (0, 0)),
        ],
        out_specs=pl.BlockSpec((ROW_TILE, DH), lambda i: (i, 0)),
        out_shape=jax.ShapeDtypeStruct((N_PAD, DH), _f32),
    )(parts, x16, w1p, wrp, b1r)


def _out_body(a2, h1r, bidr, w2, wr2, b2, wfc, bfc, out, accp, accc):
    i = pl.program_id(0)

    @pl.when(i == 0)
    def _():
        accp[...] = jnp.zeros((G, DH), _f32)
        accc[...] = jnp.zeros((G, DH), _f32)

    h2 = jnp.dot(a2[...], w2[...], preferred_element_type=_f32,
                 precision=None)
    h2 += jnp.dot(h1r[...], wr2[...], preferred_element_type=_f32,
                  precision=None)
    h2 = jnp.maximum(h2 + b2[...], 0.0)

    bid = bidr[...][0, 0]
    row_limit = jnp.where(i == N_TILES - 1, N - (N_TILES - 1) * ROW_TILE,
                          ROW_TILE)
    h2 = jnp.where(lax.broadcasted_iota(_i32, (ROW_TILE, DH), 0) < row_limit,
                   h2, 0.0)
    oh = (lax.broadcasted_iota(_i32, (G, ROW_TILE), 0) == bid[None, :]).astype(_f32)
    accp[...] += jnp.dot(oh, h2, preferred_element_type=_f32,
                         precision=None)
    accc[...] += jnp.broadcast_to(jnp.sum(oh, axis=1, keepdims=True), (G, DH))

    @pl.when(i == N_TILES - 1)
    def _():
        pooled = accp[...] / jnp.maximum(accc[...], 1.0)
        z = jnp.dot(pooled, wfc[...], preferred_element_type=_f32,
                    precision=None) + bfc[...]
        out[...] = jax.nn.sigmoid(z)


def _out_call(agg2, h1, batch3, w2, wr2, b2r, wfcp, bfcp):
    return pl.pallas_call(
        _out_body,
        grid=(N_TILES,),
        in_specs=[
            pl.BlockSpec((ROW_TILE, DH), lambda i: (i, 0)),
            pl.BlockSpec((ROW_TILE, DH), lambda i: (i, 0)),
            pl.BlockSpec((1, 1, ROW_TILE), lambda i: (i, 0, 0)),
            pl.BlockSpec((DH, DH), lambda i: (0, 0)),
            pl.BlockSpec((DH, DH), lambda i: (0, 0)),
            pl.BlockSpec((1, DH), lambda i: (0, 0)),
            pl.BlockSpec((DH, 256), lambda i: (0, 0)),
            pl.BlockSpec((1, 256), lambda i: (0, 0)),
        ],
        out_specs=pl.BlockSpec((G, 256), lambda i: (0, 0)),
        out_shape=jax.ShapeDtypeStruct((G, 256), _f32),
        scratch_shapes=[
            pltpu.VMEM((G, DH), _f32),
            pltpu.VMEM((G, DH), _f32),
        ],
    )(agg2, h1, batch3, w2, wr2, b2r, wfcp, bfcp)


@jax.jit
def kernel(x, edge_index, batch, W_rel1, b_rel1, W_root1, W_rel2, b_rel2,
           W_root2, W_fc, b_fc):
    src = edge_index[0]
    dst = edge_index[1]
    npad = E_PAD - E
    src_p = jnp.concatenate([src, (jnp.arange(npad, dtype=_i32) * 17) % N])
    dst_p = jnp.concatenate([dst, jnp.full((npad,), N, _i32)])

    x16 = jnp.zeros((N_PAD, 16), _f32).at[:N, :3].set(x)
    w1p = jnp.zeros((16, DH), _f32).at[:3].set(W_rel1)
    wrp = jnp.zeros((16, DH), _f32).at[:3].set(W_root1)
    b1r = b_rel1.reshape(1, DH)
    b2r = b_rel2.reshape(1, DH)
    wfcp = jnp.zeros((DH, 256), _f32).at[:, :138].set(W_fc)
    bfcp = jnp.zeros((1, 256), _f32).at[0, :138].set(b_fc)
    batch3 = jnp.concatenate([batch, jnp.full((N_PAD - N,), G, _i32)])
    batch3 = batch3.reshape(N_TILES, 1, ROW_TILE)
    zrosA = jnp.zeros((SHARE_A, 16), _f32)
    zrosC = jnp.zeros((SHARE_C, DH), _f32)

    parts = _agg1(x16, src_p, dst_p, zrosA)
    h1 = _h1_call(parts, x16, w1p, wrp, b1r)
    agg2 = _agg2(h1, src_p, dst_p, zrosC)
    outp = _out_call(agg2, h1, batch3, W_rel2, W_root2, b2r, wfcp, bfcp)
    return outp[:, :138]
